# Initial kernel scaffold; baseline (speedup 1.0000x reference)
import jax
import jax.numpy as jnp
from jax.experimental import pallas as pl


def kernel(categorical_ids, emb, col_ln_scale, col_ln_bias, col_W, col_b, agg_ln_scale, agg_ln_bias, agg_W1, agg_b1, agg_W2, agg_b2, head_W, head_b):
    raise NotImplementedError("write your pallas kernel here")


# speedup vs baseline: 1.0477x; 1.0477x over previous
import functools

import jax
import jax.numpy as jnp
from jax import lax
from jax.experimental import pallas as pl
from jax.experimental.pallas import tpu as pltpu
from jax.experimental.pallas import tpu_sc as plsc

N_COLS = 26
VOCAB = 100000
B = 16384
D = 100
OUT = 16
AGG = N_COLS * OUT
TOTAL = N_COLS * B

NW = 32
ROWS_PER_W = TOTAL // NW
CHUNK = 128
NCHUNK = ROWS_PER_W // CHUNK
NBUF = 4


def _sc_gather_body(emb_hbm, idx_hbm, out_hbm, idx_v, rows_v, s0, s1, s2, s3):
    sems = (s0, s1, s2, s3)
    nc = plsc.get_sparse_core_info().num_cores
    wid = lax.axis_index("s") * nc + lax.axis_index("c")
    base = wid * ROWS_PER_W
    lane = lax.iota(jnp.int32, 16)
    pltpu.sync_copy(idx_hbm.at[pl.ds(base, ROWS_PER_W)], idx_v)

    def _issue(g, b):
        def issue16(j, carry):
            vec = idx_v[pl.ds(g * CHUNK + j * 16, 16)]
            for t in range(16):
                i = jnp.sum(jnp.where(lane == t, vec, 0))
                pltpu.async_copy(
                    emb_hbm.at[pl.ds(i, 1)],
                    rows_v.at[b].at[pl.ds(j * 16 + t, 1)],
                    sems[b],
                )
            return carry

        lax.fori_loop(0, CHUNK // 16, issue16, 0)

    def _drain(b):
        pltpu.make_async_copy(
            emb_hbm.at[pl.ds(0, CHUNK)], rows_v.at[b], sems[b]
        ).wait()

    for b in range(NBUF):
        _issue(b, b)

    def outer(gg, carry):
        for b in range(NBUF):
            g = gg * NBUF + b
            _drain(b)
            pltpu.sync_copy(rows_v.at[b], out_hbm.at[pl.ds(base + g * CHUNK, CHUNK)])

            @pl.when(g + NBUF < NCHUNK)
            def _():
                _issue(g + NBUF, b)

        return carry

    lax.fori_loop(0, NCHUNK // NBUF, outer, 0)


@functools.cache
def _make_sc_gather():
    return functools.partial(
        pl.kernel,
        mesh=plsc.VectorSubcoreMesh(core_axis_name="c", subcore_axis_name="s"),
        compiler_params=pltpu.CompilerParams(needs_layout_passes=False),
        out_type=jax.ShapeDtypeStruct((TOTAL, D), jnp.float32),
        scratch_types=[
            pltpu.VMEM((ROWS_PER_W,), jnp.int32),
            pltpu.VMEM((NBUF, CHUNK, D), jnp.float32),
            pltpu.SemaphoreType.DMA,
            pltpu.SemaphoreType.DMA,
            pltpu.SemaphoreType.DMA,
            pltpu.SemaphoreType.DMA,
        ],
    )(_sc_gather_body)


BLK = 1024
_SQRT_HALF = 0.7071067811865476


def _gelu(x):
    return 0.5 * x * (1.0 + lax.erf(x * _SQRT_HALF))


def _mlp_body(x_ref, scale_ref, bias_ref, w_ref, cb_ref, p_ref,
              asc_ref, abi_ref, w1_ref, b1_ref, w2_ref, b2_ref, hw_ref, hb_ref,
              logits_ref, feats_ref):
    scale = scale_ref[...]
    bias = bias_ref[...]
    cb = cb_ref[...]
    acc = jnp.zeros((BLK, AGG), jnp.float32)
    for c in range(N_COLS):
        xc = x_ref[c]
        s1 = jnp.sum(xc, axis=1, keepdims=True)
        s2 = jnp.sum(xc * xc, axis=1, keepdims=True)
        mean = s1 * (1.0 / D)
        var = s2 * (1.0 / D) - mean * mean
        xn = (xc - mean) * lax.rsqrt(var + 1e-5)
        y = xn * lax.slice(scale, (c, 0), (c + 1, D)) + lax.slice(bias, (c, 0), (c + 1, D))
        g = jnp.dot(y, w_ref[c], preferred_element_type=jnp.float32)
        g = _gelu(g + lax.slice(cb, (c, 0), (c + 1, OUT)))
        acc = acc + jnp.dot(g.astype(jnp.bfloat16), p_ref[c],
                            preferred_element_type=jnp.float32)
    m2 = jnp.mean(acc, axis=1, keepdims=True)
    v2 = jnp.mean(acc * acc, axis=1, keepdims=True) - m2 * m2
    h = (acc - m2) * lax.rsqrt(v2 + 1e-5) * asc_ref[...] + abi_ref[...]
    h1 = jnp.dot(h.astype(jnp.bfloat16), w1_ref[...],
                 preferred_element_type=jnp.float32) + b1_ref[...]
    h1 = _gelu(h1)
    feats = jnp.dot(h1, w2_ref[...], preferred_element_type=jnp.float32) + b2_ref[...]
    logits = jnp.dot(feats, hw_ref[...], preferred_element_type=jnp.float32) + hb_ref[...]
    feats_ref[...] = feats
    logits_ref[...] = logits


def _full(shape):
    return pl.BlockSpec(shape, lambda i: (0,) * len(shape))


_mlp = pl.pallas_call(
    _mlp_body,
    grid=(B // BLK,),
    in_specs=[
        pl.BlockSpec((N_COLS, BLK, D), lambda i: (0, i, 0)),
        _full((N_COLS, D)),
        _full((N_COLS, D)),
        _full((N_COLS, D, OUT)),
        _full((N_COLS, OUT)),
        _full((N_COLS, OUT, AGG)),
        _full((1, AGG)),
        _full((1, AGG)),
        _full((AGG, AGG)),
        _full((1, AGG)),
        _full((AGG, OUT)),
        _full((1, OUT)),
        _full((OUT, 2)),
        _full((1, 2)),
    ],
    out_specs=[
        pl.BlockSpec((BLK, 2), lambda i: (i, 0)),
        pl.BlockSpec((BLK, OUT), lambda i: (i, 0)),
    ],
    out_shape=[
        jax.ShapeDtypeStruct((B, 2), jnp.float32),
        jax.ShapeDtypeStruct((B, OUT), jnp.float32),
    ],
    compiler_params=pltpu.CompilerParams(vmem_limit_bytes=100 * 1024 * 1024),
)


def kernel(categorical_ids, emb, col_ln_scale, col_ln_bias, col_W, col_b,
           agg_ln_scale, agg_ln_bias, agg_W1, agg_b1, agg_W2, agg_b2,
           head_W, head_b):
    emb_flat = emb.reshape(N_COLS * VOCAB, D)
    offs = (jnp.arange(N_COLS, dtype=jnp.int32) * VOCAB)[:, None]
    idx = (categorical_ids + offs).reshape(TOTAL)
    gathered = _make_sc_gather()(emb_flat, idx)
    x3d = gathered.reshape(N_COLS, B, D)

    p = (jnp.eye(N_COLS, dtype=jnp.bfloat16)[:, None, :, None]
         * jnp.eye(OUT, dtype=jnp.bfloat16)[None, :, None, :]).reshape(
             N_COLS, OUT, AGG)

    logits, feats = _mlp(
        x3d, col_ln_scale, col_ln_bias, col_W, col_b, p,
        agg_ln_scale.reshape(1, AGG), agg_ln_bias.reshape(1, AGG),
        agg_W1.astype(jnp.bfloat16), agg_b1.reshape(1, AGG),
        agg_W2, agg_b2.reshape(1, OUT),
        head_W, head_b.reshape(1, 2),
    )
    return (logits, feats)


# speedup vs baseline: 1.0481x; 1.0004x over previous
import functools

import jax
import jax.numpy as jnp
from jax import lax
from jax.experimental import pallas as pl
from jax.experimental.pallas import tpu as pltpu
from jax.experimental.pallas import tpu_sc as plsc

N_COLS = 26
VOCAB = 100000
B = 16384
D = 100
OUT = 16
AGG = N_COLS * OUT
TOTAL = N_COLS * B

NW = 32
ROWS_PER_W = TOTAL // NW
CHUNK = 128
NCHUNK = ROWS_PER_W // CHUNK
NBUF = 4


def _sc_gather_body(emb_hbm, idx_hbm, out_hbm, idx_v, rows_v, s0, s1, s2, s3):
    sems = (s0, s1, s2, s3)
    nc = plsc.get_sparse_core_info().num_cores
    wid = lax.axis_index("s") * nc + lax.axis_index("c")
    base = wid * ROWS_PER_W
    pltpu.sync_copy(idx_hbm.at[pl.ds(base, ROWS_PER_W)], idx_v)

    def _issue(g, b):
        def issue16(j, carry):
            vec = idx_v[pl.ds(g * CHUNK + j * 16, 16)]
            for t in range(16):
                i = vec[t]
                pltpu.async_copy(
                    emb_hbm.at[pl.ds(i, 1)],
                    rows_v.at[b].at[pl.ds(j * 16 + t, 1)],
                    sems[b],
                )
            return carry

        lax.fori_loop(0, CHUNK // 16, issue16, 0)

    def _drain(b):
        pltpu.make_async_copy(
            emb_hbm.at[pl.ds(0, CHUNK)], rows_v.at[b], sems[b]
        ).wait()

    for b in range(NBUF):
        _issue(b, b)

    def outer(gg, carry):
        for b in range(NBUF):
            g = gg * NBUF + b
            _drain(b)
            pltpu.sync_copy(rows_v.at[b], out_hbm.at[pl.ds(base + g * CHUNK, CHUNK)])

            @pl.when(g + NBUF < NCHUNK)
            def _():
                _issue(g + NBUF, b)

        return carry

    lax.fori_loop(0, NCHUNK // NBUF, outer, 0)


@functools.cache
def _make_sc_gather():
    return functools.partial(
        pl.kernel,
        mesh=plsc.VectorSubcoreMesh(core_axis_name="c", subcore_axis_name="s"),
        out_type=jax.ShapeDtypeStruct((TOTAL, D), jnp.float32),
        scratch_types=[
            pltpu.VMEM((ROWS_PER_W,), jnp.int32),
            pltpu.VMEM((NBUF, CHUNK, D), jnp.float32),
            pltpu.SemaphoreType.DMA,
            pltpu.SemaphoreType.DMA,
            pltpu.SemaphoreType.DMA,
            pltpu.SemaphoreType.DMA,
        ],
    )(_sc_gather_body)


BLK = 1024
_SQRT_HALF = 0.7071067811865476


def _gelu(x):
    return 0.5 * x * (1.0 + lax.erf(x * _SQRT_HALF))


def _mlp_body(x_ref, scale_ref, bias_ref, w_ref, cb_ref, p_ref,
              asc_ref, abi_ref, w1_ref, b1_ref, w2_ref, b2_ref, hw_ref, hb_ref,
              logits_ref, feats_ref):
    scale = scale_ref[...]
    bias = bias_ref[...]
    cb = cb_ref[...]
    acc = jnp.zeros((BLK, AGG), jnp.float32)
    for c in range(N_COLS):
        xc = x_ref[c]
        s1 = jnp.sum(xc, axis=1, keepdims=True)
        s2 = jnp.sum(xc * xc, axis=1, keepdims=True)
        mean = s1 * (1.0 / D)
        var = s2 * (1.0 / D) - mean * mean
        xn = (xc - mean) * lax.rsqrt(var + 1e-5)
        y = xn * lax.slice(scale, (c, 0), (c + 1, D)) + lax.slice(bias, (c, 0), (c + 1, D))
        g = jnp.dot(y, w_ref[c], preferred_element_type=jnp.float32)
        g = _gelu(g + lax.slice(cb, (c, 0), (c + 1, OUT)))
        acc = acc + jnp.dot(g.astype(jnp.bfloat16), p_ref[c],
                            preferred_element_type=jnp.float32)
    m2 = jnp.mean(acc, axis=1, keepdims=True)
    v2 = jnp.mean(acc * acc, axis=1, keepdims=True) - m2 * m2
    h = (acc - m2) * lax.rsqrt(v2 + 1e-5) * asc_ref[...] + abi_ref[...]
    h1 = jnp.dot(h.astype(jnp.bfloat16), w1_ref[...],
                 preferred_element_type=jnp.float32) + b1_ref[...]
    h1 = _gelu(h1)
    feats = jnp.dot(h1, w2_ref[...], preferred_element_type=jnp.float32) + b2_ref[...]
    logits = jnp.dot(feats, hw_ref[...], preferred_element_type=jnp.float32) + hb_ref[...]
    feats_ref[...] = feats
    logits_ref[...] = logits


def _full(shape):
    return pl.BlockSpec(shape, lambda i: (0,) * len(shape))


_mlp = pl.pallas_call(
    _mlp_body,
    grid=(B // BLK,),
    in_specs=[
        pl.BlockSpec((N_COLS, BLK, D), lambda i: (0, i, 0)),
        _full((N_COLS, D)),
        _full((N_COLS, D)),
        _full((N_COLS, D, OUT)),
        _full((N_COLS, OUT)),
        _full((N_COLS, OUT, AGG)),
        _full((1, AGG)),
        _full((1, AGG)),
        _full((AGG, AGG)),
        _full((1, AGG)),
        _full((AGG, OUT)),
        _full((1, OUT)),
        _full((OUT, 2)),
        _full((1, 2)),
    ],
    out_specs=[
        pl.BlockSpec((BLK, 2), lambda i: (i, 0)),
        pl.BlockSpec((BLK, OUT), lambda i: (i, 0)),
    ],
    out_shape=[
        jax.ShapeDtypeStruct((B, 2), jnp.float32),
        jax.ShapeDtypeStruct((B, OUT), jnp.float32),
    ],
    compiler_params=pltpu.CompilerParams(vmem_limit_bytes=100 * 1024 * 1024),
)


def kernel(categorical_ids, emb, col_ln_scale, col_ln_bias, col_W, col_b,
           agg_ln_scale, agg_ln_bias, agg_W1, agg_b1, agg_W2, agg_b2,
           head_W, head_b):
    emb_flat = emb.reshape(N_COLS * VOCAB, D)
    offs = (jnp.arange(N_COLS, dtype=jnp.int32) * VOCAB)[:, None]
    idx = (categorical_ids + offs).reshape(TOTAL)
    gathered = _make_sc_gather()(emb_flat, idx)
    x3d = gathered.reshape(N_COLS, B, D)

    p = (jnp.eye(N_COLS, dtype=jnp.bfloat16)[:, None, :, None]
         * jnp.eye(OUT, dtype=jnp.bfloat16)[None, :, None, :]).reshape(
             N_COLS, OUT, AGG)

    logits, feats = _mlp(
        x3d, col_ln_scale, col_ln_bias, col_W, col_b, p,
        agg_ln_scale.reshape(1, AGG), agg_ln_bias.reshape(1, AGG),
        agg_W1.astype(jnp.bfloat16), agg_b1.reshape(1, AGG),
        agg_W2, agg_b2.reshape(1, OUT),
        head_W, head_b.reshape(1, 2),
    )
    return (logits, feats)


# speedup vs baseline: 2.9363x; 2.8017x over previous
import functools

import jax
import jax.numpy as jnp
from jax import lax
from jax.experimental import pallas as pl
from jax.experimental.pallas import tpu as pltpu
from jax.experimental.pallas import tpu_sc as plsc

N_COLS = 26
VOCAB = 100000
B = 16384
D = 100
DP = 128
OUT = 16
AGG = N_COLS * OUT
TOTAL = N_COLS * B
GROWS = N_COLS * DP

NW = 32
BCHUNK = 8192
NB = B // BCHUNK


def _sc_gather_body(emb_t, ids_flat, out_hbm, slab_v, idx_v, out_v, sem):
    nc = plsc.get_sparse_core_info().num_cores
    wid = lax.axis_index("s") * nc + lax.axis_index("c")
    zeros16 = jnp.zeros((16,), jnp.int32)

    for c in range(N_COLS):
        for k in range(4):
            d = k * NW + wid

            @pl.when(d < D)
            def _():
                pltpu.sync_copy(emb_t.at[c].at[pl.ds(d, 1)], slab_v)
                for bc in range(NB):
                    pltpu.sync_copy(
                        ids_flat.at[pl.ds(c * B + bc * BCHUNK, BCHUNK)], idx_v)

                    def select16(m, carry):
                        ids16 = idx_v[pl.ds(m * 16, 16)]
                        vals = plsc.load_gather(slab_v, [zeros16, ids16])
                        out_v[pl.ds(m * 16, 16)] = vals
                        return carry

                    lax.fori_loop(0, BCHUNK // 16, select16, 0)
                    row = c * DP + d
                    pltpu.sync_copy(
                        out_v,
                        out_hbm.at[row].at[pl.ds(bc * BCHUNK, BCHUNK)])


@functools.cache
def _make_sc_gather():
    return functools.partial(
        pl.kernel,
        mesh=plsc.VectorSubcoreMesh(core_axis_name="c", subcore_axis_name="s"),
        compiler_params=pltpu.CompilerParams(needs_layout_passes=False),
        out_type=jax.ShapeDtypeStruct((GROWS, B), jnp.float32),
        scratch_types=[
            pltpu.VMEM((1, VOCAB), jnp.float32),
            pltpu.VMEM((BCHUNK,), jnp.int32),
            pltpu.VMEM((BCHUNK,), jnp.float32),
            pltpu.SemaphoreType.DMA,
        ],
    )(_sc_gather_body)


BLK = 1024
_SQRT_HALF = 0.7071067811865476


def _gelu(x):
    return 0.5 * x * (1.0 + lax.erf(x * _SQRT_HALF))


def _mlp_body(x_ref, w_ref, cb_ref, p_ref,
              asc_ref, abi_ref, w1_ref, b1_ref, w2_ref, b2_ref, hw_ref, hb_ref,
              logits_ref, feats_ref):
    cb = cb_ref[...]
    acc = jnp.zeros((BLK, AGG), jnp.float32)
    dn = (((0,), (0,)), ((), ()))
    for c in range(N_COLS):
        xc = x_ref[pl.ds(c * DP, D), :]
        s1 = jnp.sum(xc, axis=0, keepdims=True)
        s2 = jnp.sum(xc * xc, axis=0, keepdims=True)
        mean = s1 * (1.0 / D)
        var = s2 * (1.0 / D) - mean * mean
        xn = (xc - mean) * lax.rsqrt(var + 1e-5)
        g = lax.dot_general(xn, w_ref[c], dn,
                            preferred_element_type=jnp.float32)
        g = _gelu(g + lax.slice(cb, (c, 0), (c + 1, OUT)))
        acc = acc + jnp.dot(g.astype(jnp.bfloat16), p_ref[c],
                            preferred_element_type=jnp.float32)
    m2 = jnp.mean(acc, axis=1, keepdims=True)
    v2 = jnp.mean(acc * acc, axis=1, keepdims=True) - m2 * m2
    h = (acc - m2) * lax.rsqrt(v2 + 1e-5) * asc_ref[...] + abi_ref[...]
    h1 = jnp.dot(h.astype(jnp.bfloat16), w1_ref[...],
                 preferred_element_type=jnp.float32) + b1_ref[...]
    h1 = _gelu(h1)
    feats = jnp.dot(h1, w2_ref[...], preferred_element_type=jnp.float32) + b2_ref[...]
    logits = jnp.dot(feats, hw_ref[...], preferred_element_type=jnp.float32) + hb_ref[...]
    feats_ref[...] = feats
    logits_ref[...] = logits


def _full(shape):
    return pl.BlockSpec(shape, lambda i: (0,) * len(shape))


_mlp = pl.pallas_call(
    _mlp_body,
    grid=(B // BLK,),
    in_specs=[
        pl.BlockSpec((GROWS, BLK), lambda i: (0, i)),
        _full((N_COLS, D, OUT)),
        _full((N_COLS, OUT)),
        _full((N_COLS, OUT, AGG)),
        _full((1, AGG)),
        _full((1, AGG)),
        _full((AGG, AGG)),
        _full((1, AGG)),
        _full((AGG, OUT)),
        _full((1, OUT)),
        _full((OUT, 2)),
        _full((1, 2)),
    ],
    out_specs=[
        pl.BlockSpec((BLK, 2), lambda i: (i, 0)),
        pl.BlockSpec((BLK, OUT), lambda i: (i, 0)),
    ],
    out_shape=[
        jax.ShapeDtypeStruct((B, 2), jnp.float32),
        jax.ShapeDtypeStruct((B, OUT), jnp.float32),
    ],
    compiler_params=pltpu.CompilerParams(vmem_limit_bytes=100 * 1024 * 1024),
)


def kernel(categorical_ids, emb, col_ln_scale, col_ln_bias, col_W, col_b,
           agg_ln_scale, agg_ln_bias, agg_W1, agg_b1, agg_W2, agg_b2,
           head_W, head_b):
    emb_t = jnp.transpose(emb, (0, 2, 1))
    ids_flat = categorical_ids.reshape(TOTAL)
    gathered = _make_sc_gather()(emb_t, ids_flat)

    w_fold = col_ln_scale[:, :, None] * col_W
    cb_fold = jnp.einsum("cd,cdo->co", col_ln_bias, col_W) + col_b
    p = (jnp.eye(N_COLS, dtype=jnp.bfloat16)[:, None, :, None]
         * jnp.eye(OUT, dtype=jnp.bfloat16)[None, :, None, :]).reshape(
             N_COLS, OUT, AGG)

    logits, feats = _mlp(
        gathered, w_fold, cb_fold, p,
        agg_ln_scale.reshape(1, AGG), agg_ln_bias.reshape(1, AGG),
        agg_W1.astype(jnp.bfloat16), agg_b1.reshape(1, AGG),
        agg_W2, agg_b2.reshape(1, OUT),
        head_W, head_b.reshape(1, 2),
    )
    return (logits, feats)


# speedup vs baseline: 3.8225x; 1.3018x over previous
import functools

import jax
import jax.numpy as jnp
from jax import lax
from jax.experimental import pallas as pl
from jax.experimental.pallas import tpu as pltpu
from jax.experimental.pallas import tpu_sc as plsc

N_COLS = 26
VOCAB = 100000
B = 16384
D = 100
DP = 128
OUT = 16
AGG = N_COLS * OUT
TOTAL = N_COLS * B
GROWS = N_COLS * DP

NW = 32
BCHUNK = 8192
NB = B // BCHUNK


def _sc_gather_body(emb_t, ids_flat, out_hbm, slab_v, idx_v, out_v, sem):
    nc = plsc.get_sparse_core_info().num_cores
    wid = lax.axis_index("s") * nc + lax.axis_index("c")
    zeros16 = jnp.zeros((16,), jnp.int32)

    UNROLL = 8

    def col_body(c, carry0):
        pltpu.sync_copy(ids_flat.at[pl.ds(c * B, B)], idx_v)

        def k_body(k, carry1):
            d = k * NW + wid

            @pl.when(d < D)
            def _():
                pltpu.sync_copy(emb_t.at[c].at[pl.ds(d, 1)], slab_v)
                for bc in range(NB):

                    def select(m, carry):
                        for u in range(UNROLL):
                            off = (m * UNROLL + u) * 16
                            ids16 = idx_v[pl.ds(bc * BCHUNK + off, 16)]
                            vals = plsc.load_gather(slab_v, [zeros16, ids16])
                            out_v[pl.ds(off, 16)] = vals
                        return carry

                    lax.fori_loop(0, BCHUNK // 16 // UNROLL, select, 0)
                    row = c * DP + d
                    pltpu.sync_copy(
                        out_v,
                        out_hbm.at[row].at[pl.ds(bc * BCHUNK, BCHUNK)])

            return carry1

        lax.fori_loop(0, 4, k_body, 0)
        return carry0

    lax.fori_loop(0, N_COLS, col_body, 0)


@functools.cache
def _make_sc_gather():
    return functools.partial(
        pl.kernel,
        mesh=plsc.VectorSubcoreMesh(core_axis_name="c", subcore_axis_name="s"),
        compiler_params=pltpu.CompilerParams(needs_layout_passes=False),
        out_type=jax.ShapeDtypeStruct((GROWS, B), jnp.float32),
        scratch_types=[
            pltpu.VMEM((1, VOCAB), jnp.float32),
            pltpu.VMEM((B,), jnp.int32),
            pltpu.VMEM((BCHUNK,), jnp.float32),
            pltpu.SemaphoreType.DMA,
        ],
    )(_sc_gather_body)


BLK = 1024
_SQRT_HALF = 0.7071067811865476


def _gelu(x):
    return 0.5 * x * (1.0 + lax.erf(x * _SQRT_HALF))


def _mlp_body(x_ref, w_ref, cb_ref, p_ref,
              asc_ref, abi_ref, w1_ref, b1_ref, w2_ref, b2_ref, hw_ref, hb_ref,
              logits_ref, feats_ref):
    cb = cb_ref[...]
    acc = jnp.zeros((BLK, AGG), jnp.float32)
    dn = (((0,), (0,)), ((), ()))
    for c in range(N_COLS):
        xc = x_ref[pl.ds(c * DP, D), :]
        s1 = jnp.sum(xc, axis=0, keepdims=True)
        s2 = jnp.sum(xc * xc, axis=0, keepdims=True)
        mean = s1 * (1.0 / D)
        var = s2 * (1.0 / D) - mean * mean
        xn = (xc - mean) * lax.rsqrt(var + 1e-5)
        g = lax.dot_general(xn, w_ref[c], dn,
                            preferred_element_type=jnp.float32)
        g = _gelu(g + lax.slice(cb, (c, 0), (c + 1, OUT)))
        acc = acc + jnp.dot(g.astype(jnp.bfloat16), p_ref[c],
                            preferred_element_type=jnp.float32)
    m2 = jnp.mean(acc, axis=1, keepdims=True)
    v2 = jnp.mean(acc * acc, axis=1, keepdims=True) - m2 * m2
    h = (acc - m2) * lax.rsqrt(v2 + 1e-5) * asc_ref[...] + abi_ref[...]
    h1 = jnp.dot(h.astype(jnp.bfloat16), w1_ref[...],
                 preferred_element_type=jnp.float32) + b1_ref[...]
    h1 = _gelu(h1)
    feats = jnp.dot(h1, w2_ref[...], preferred_element_type=jnp.float32) + b2_ref[...]
    logits = jnp.dot(feats, hw_ref[...], preferred_element_type=jnp.float32) + hb_ref[...]
    feats_ref[...] = feats
    logits_ref[...] = logits


def _full(shape):
    return pl.BlockSpec(shape, lambda i: (0,) * len(shape))


_mlp = pl.pallas_call(
    _mlp_body,
    grid=(B // BLK,),
    in_specs=[
        pl.BlockSpec((GROWS, BLK), lambda i: (0, i)),
        _full((N_COLS, D, OUT)),
        _full((N_COLS, OUT)),
        _full((N_COLS, OUT, AGG)),
        _full((1, AGG)),
        _full((1, AGG)),
        _full((AGG, AGG)),
        _full((1, AGG)),
        _full((AGG, OUT)),
        _full((1, OUT)),
        _full((OUT, 2)),
        _full((1, 2)),
    ],
    out_specs=[
        pl.BlockSpec((BLK, 2), lambda i: (i, 0)),
        pl.BlockSpec((BLK, OUT), lambda i: (i, 0)),
    ],
    out_shape=[
        jax.ShapeDtypeStruct((B, 2), jnp.float32),
        jax.ShapeDtypeStruct((B, OUT), jnp.float32),
    ],
    compiler_params=pltpu.CompilerParams(vmem_limit_bytes=100 * 1024 * 1024),
)


def kernel(categorical_ids, emb, col_ln_scale, col_ln_bias, col_W, col_b,
           agg_ln_scale, agg_ln_bias, agg_W1, agg_b1, agg_W2, agg_b2,
           head_W, head_b):
    emb_t = jnp.transpose(emb, (0, 2, 1))
    ids_flat = categorical_ids.reshape(TOTAL)
    gathered = _make_sc_gather()(emb_t, ids_flat)

    w_fold = col_ln_scale[:, :, None] * col_W
    cb_fold = jnp.einsum("cd,cdo->co", col_ln_bias, col_W) + col_b
    p = (jnp.eye(N_COLS, dtype=jnp.bfloat16)[:, None, :, None]
         * jnp.eye(OUT, dtype=jnp.bfloat16)[None, :, None, :]).reshape(
             N_COLS, OUT, AGG)

    logits, feats = _mlp(
        gathered, w_fold, cb_fold, p,
        agg_ln_scale.reshape(1, AGG), agg_ln_bias.reshape(1, AGG),
        agg_W1.astype(jnp.bfloat16), agg_b1.reshape(1, AGG),
        agg_W2, agg_b2.reshape(1, OUT),
        head_W, head_b.reshape(1, 2),
    )
    return (logits, feats)
